# baseline (speedup 1.0000x reference)
import jax
import jax.numpy as jnp
from jax import lax
from jax.experimental import pallas as pl
from jax.experimental.pallas import tpu as pltpu
from jax.experimental.pallas import tpu_sc as plsc

_B = 8192
_D = 2048
_E = 16
_M_BLK = 1024
_CHUNKS = 2
_CB = _B // _CHUNKS

_NC = 2
_NS = 16
_NW = _NC * _NS
_ROWS_PER_W = _CB // _NW
_GROUPS = _ROWS_PER_W // 16


def _matmul_body(x_ref, w_ref, out_t_ref):
    out_t_ref[...] = lax.dot_general(
        w_ref[...], x_ref[...],
        dimension_numbers=(((1,), (1,)), ((), ())),
        preferred_element_type=jnp.float32)


@jax.jit
def _logits_call(x, w):
    return pl.pallas_call(
        _matmul_body,
        grid=(_CB // _M_BLK,),
        in_specs=[
            pl.BlockSpec((_M_BLK, _D), lambda i: (i, 0)),
            pl.BlockSpec((_E, _D), lambda i: (0, 0)),
        ],
        out_specs=pl.BlockSpec((_E, _M_BLK), lambda i: (0, i)),
        out_shape=jax.ShapeDtypeStruct((_E, _CB), jnp.float32),
        compiler_params=pltpu.CompilerParams(
            dimension_semantics=("arbitrary",)),
    )(x, w)


def _gate_body(logits_hbm, out_hbm, logits_v, out_v):
    wid = lax.axis_index("s") * _NC + lax.axis_index("c")
    base = wid * _ROWS_PER_W
    pltpu.sync_copy(logits_hbm.at[:, pl.ds(base, _ROWS_PER_W)], logits_v)

    def group(g, carry):
        sl = pl.ds(g * 16, 16)

        def expert(e, st):
            m1, m2, i1, i2 = st
            v = logits_v[e, sl]
            ev = jnp.full((16,), e, jnp.int32)
            gt1 = v > m1
            gt2 = v > m2
            m2 = jnp.where(gt1, m1, jnp.where(gt2, v, m2))
            i2 = jnp.where(gt1, i1, jnp.where(gt2, ev, i2))
            m1 = jnp.where(gt1, v, m1)
            i1 = jnp.where(gt1, ev, i1)
            return (m1, m2, i1, i2)

        m1, m2, i1, i2 = lax.fori_loop(
            0, _E, expert,
            (jnp.full((16,), -jnp.inf, jnp.float32),
             jnp.full((16,), -jnp.inf, jnp.float32),
             jnp.zeros((16,), jnp.int32),
             jnp.zeros((16,), jnp.int32)))
        w1 = 1.0 / (1.0 + jnp.exp(m2 - m1))
        w2 = 1.0 - w1
        out_v[0, sl] = w1
        out_v[1, sl] = w2
        out_v[2, sl] = plsc.bitcast(i1, jnp.float32)
        out_v[3, sl] = plsc.bitcast(i2, jnp.float32)
        return carry

    lax.fori_loop(0, _GROUPS, group, 0)

    pltpu.sync_copy(out_v, out_hbm.at[:, pl.ds(base, _ROWS_PER_W)])


@jax.jit
def _gate_call(logits_t):
    f = pl.kernel(
        _gate_body,
        mesh=plsc.VectorSubcoreMesh(
            core_axis_name="c", subcore_axis_name="s"),
        out_type=jax.ShapeDtypeStruct((8, _CB), jnp.float32),
        scratch_types=[
            pltpu.VMEM((_E, _ROWS_PER_W), jnp.float32),
            pltpu.VMEM((8, _ROWS_PER_W), jnp.float32),
        ],
        compiler_params=pltpu.CompilerParams(
            needs_layout_passes=False, use_tc_tiling_on_sc=True),
    )
    return f(logits_t)


def kernel(x, W_gate, W_noise):
    lts = [_logits_call(x[c * _CB:(c + 1) * _CB], W_gate)
           for c in range(_CHUNKS)]
    packs = [_gate_call(lt) for lt in lts]
    clean_logits = jnp.concatenate([lt.T for lt in lts], axis=0)
    pack = jnp.concatenate(packs, axis=1)
    combined_weights = pack[0:2, :].T
    top_k_indices = lax.bitcast_convert_type(pack[2:4, :].T, jnp.int32)
    return (combined_weights, top_k_indices, clean_logits)


# speedup vs baseline: 1.8473x; 1.8473x over previous
import jax
import jax.numpy as jnp
from jax import lax
from jax.experimental import pallas as pl
from jax.experimental.pallas import tpu as pltpu
from jax.experimental.pallas import tpu_sc as plsc

_B = 8192
_D = 2048
_E = 16
_M_BLK = 2048
_CHUNKS = 1
_CB = _B // _CHUNKS

_NC = 2
_NS = 16
_NW = _NC * _NS
_ROWS_PER_W = _CB // _NW
_GROUPS = _ROWS_PER_W // 16


def _matmul_body(x_ref, w_ref, out_t_ref):
    out_t_ref[...] = lax.dot_general(
        w_ref[...], x_ref[...],
        dimension_numbers=(((1,), (1,)), ((), ())),
        preferred_element_type=jnp.float32)


@jax.jit
def _logits_call(x, w):
    return pl.pallas_call(
        _matmul_body,
        grid=(_CB // _M_BLK,),
        in_specs=[
            pl.BlockSpec((_M_BLK, _D), lambda i: (i, 0)),
            pl.BlockSpec((_E, _D), lambda i: (0, 0)),
        ],
        out_specs=pl.BlockSpec((_E, _M_BLK), lambda i: (0, i)),
        out_shape=jax.ShapeDtypeStruct((_E, _CB), jnp.float32),
        compiler_params=pltpu.CompilerParams(
            dimension_semantics=("arbitrary",)),
    )(x, w)


def _gate_body(logits_hbm, out_hbm, logits_v, out_v):
    wid = lax.axis_index("s") * _NC + lax.axis_index("c")
    base = wid * _ROWS_PER_W
    pltpu.sync_copy(logits_hbm.at[:, pl.ds(base, _ROWS_PER_W)], logits_v)

    def group(g, carry):
        sl = pl.ds(g * 16, 16)

        def expert(e, st):
            m1, m2, i1, i2 = st
            v = logits_v[e, sl]
            ev = jnp.full((16,), e, jnp.int32)
            gt1 = v > m1
            gt2 = v > m2
            m2 = jnp.where(gt1, m1, jnp.where(gt2, v, m2))
            i2 = jnp.where(gt1, i1, jnp.where(gt2, ev, i2))
            m1 = jnp.where(gt1, v, m1)
            i1 = jnp.where(gt1, ev, i1)
            return (m1, m2, i1, i2)

        m1, m2, i1, i2 = lax.fori_loop(
            0, _E, expert,
            (jnp.full((16,), -jnp.inf, jnp.float32),
             jnp.full((16,), -jnp.inf, jnp.float32),
             jnp.zeros((16,), jnp.int32),
             jnp.zeros((16,), jnp.int32)))
        w1 = 1.0 / (1.0 + jnp.exp(m2 - m1))
        w2 = 1.0 - w1
        out_v[0, sl] = w1
        out_v[1, sl] = w2
        out_v[2, sl] = plsc.bitcast(i1, jnp.float32)
        out_v[3, sl] = plsc.bitcast(i2, jnp.float32)
        return carry

    lax.fori_loop(0, _GROUPS, group, 0)

    pltpu.sync_copy(out_v, out_hbm.at[:, pl.ds(base, _ROWS_PER_W)])


@jax.jit
def _gate_call(logits_t):
    f = pl.kernel(
        _gate_body,
        mesh=plsc.VectorSubcoreMesh(
            core_axis_name="c", subcore_axis_name="s"),
        out_type=jax.ShapeDtypeStruct((8, _CB), jnp.float32),
        scratch_types=[
            pltpu.VMEM((_E, _ROWS_PER_W), jnp.float32),
            pltpu.VMEM((8, _ROWS_PER_W), jnp.float32),
        ],
        compiler_params=pltpu.CompilerParams(
            needs_layout_passes=False, use_tc_tiling_on_sc=True),
    )
    return f(logits_t)


def kernel(x, W_gate, W_noise):
    logits_t = _logits_call(x, W_gate)
    pack = _gate_call(logits_t)
    clean_logits = logits_t.T
    combined_weights = pack[0:2, :].T
    top_k_indices = lax.bitcast_convert_type(pack[2:4, :].T, jnp.int32)
    return (combined_weights, top_k_indices, clean_logits)


# speedup vs baseline: 1.9759x; 1.0696x over previous
import jax
import jax.numpy as jnp
from jax import lax
from jax.experimental import pallas as pl
from jax.experimental.pallas import tpu as pltpu
from jax.experimental.pallas import tpu_sc as plsc

_B = 8192
_D = 2048
_E = 16
_M_BLK = 1024
_CHUNKS = 1
_CB = _B // _CHUNKS

_NC = 2
_NS = 16
_NW = _NC * _NS
_ROWS_PER_W = _CB // _NW
_GROUPS = _ROWS_PER_W // 16


def _matmul_body(x_ref, w_ref, out_t_ref):
    out_t_ref[...] = lax.dot_general(
        w_ref[...], x_ref[...],
        dimension_numbers=(((1,), (1,)), ((), ())),
        preferred_element_type=jnp.float32)


@jax.jit
def _logits_call(x, w):
    return pl.pallas_call(
        _matmul_body,
        grid=(_CB // _M_BLK,),
        in_specs=[
            pl.BlockSpec((_M_BLK, _D), lambda i: (i, 0)),
            pl.BlockSpec((_E, _D), lambda i: (0, 0)),
        ],
        out_specs=pl.BlockSpec((_E, _M_BLK), lambda i: (0, i)),
        out_shape=jax.ShapeDtypeStruct((_E, _CB), jnp.float32),
        compiler_params=pltpu.CompilerParams(
            dimension_semantics=("arbitrary",)),
    )(x, w)


def _gate_body(logits_hbm, out_hbm, logits_v, out_v):
    wid = lax.axis_index("s") * _NC + lax.axis_index("c")
    base = wid * _ROWS_PER_W
    pltpu.sync_copy(logits_hbm.at[:, pl.ds(base, _ROWS_PER_W)], logits_v)

    def group(g, carry):
        sl = pl.ds(g * 16, 16)

        def expert(e, st):
            m1, m2, i1, i2 = st
            v = logits_v[e, sl]
            ev = jnp.full((16,), e, jnp.int32)
            gt1 = v > m1
            gt2 = v > m2
            m2 = jnp.where(gt1, m1, jnp.where(gt2, v, m2))
            i2 = jnp.where(gt1, i1, jnp.where(gt2, ev, i2))
            m1 = jnp.where(gt1, v, m1)
            i1 = jnp.where(gt1, ev, i1)
            return (m1, m2, i1, i2)

        m1, m2, i1, i2 = lax.fori_loop(
            0, _E, expert,
            (jnp.full((16,), -jnp.inf, jnp.float32),
             jnp.full((16,), -jnp.inf, jnp.float32),
             jnp.zeros((16,), jnp.int32),
             jnp.zeros((16,), jnp.int32)))
        w1 = 1.0 / (1.0 + jnp.exp(m2 - m1))
        w2 = 1.0 - w1
        out_v[0, sl] = w1
        out_v[1, sl] = w2
        out_v[2, sl] = plsc.bitcast(i1, jnp.float32)
        out_v[3, sl] = plsc.bitcast(i2, jnp.float32)
        return carry

    lax.fori_loop(0, _GROUPS, group, 0)

    pltpu.sync_copy(out_v, out_hbm.at[:, pl.ds(base, _ROWS_PER_W)])


@jax.jit
def _gate_call(logits_t):
    f = pl.kernel(
        _gate_body,
        mesh=plsc.VectorSubcoreMesh(
            core_axis_name="c", subcore_axis_name="s"),
        out_type=jax.ShapeDtypeStruct((8, _CB), jnp.float32),
        scratch_types=[
            pltpu.VMEM((_E, _ROWS_PER_W), jnp.float32),
            pltpu.VMEM((8, _ROWS_PER_W), jnp.float32),
        ],
        compiler_params=pltpu.CompilerParams(
            needs_layout_passes=False, use_tc_tiling_on_sc=True),
    )
    return f(logits_t)


def kernel(x, W_gate, W_noise):
    logits_t = _logits_call(x, W_gate)
    pack = _gate_call(logits_t)
    clean_logits = logits_t.T
    combined_weights = pack[0:2, :].T
    top_k_indices = lax.bitcast_convert_type(pack[2:4, :].T, jnp.int32)
    return (combined_weights, top_k_indices, clean_logits)
